# baseline (speedup 1.0000x reference)
import functools

import jax
import jax.numpy as jnp
from jax.experimental import pallas as pl
from jax.experimental.pallas import tpu as pltpu


def _relation_kernel(p_hbm, o_hbm, w_hbm, b_hbm, out_hbm,
                     p_v, o_v, w_v, b_v, r_v, sem, *,
                     f_num, n_p, n_o, d, in_chunk, out_chunk):
    copies = []
    idx = 0

    def start_copy(src, dst):
        nonlocal idx
        c = pltpu.make_async_copy(src, dst, sem.at[idx])
        c.start()
        copies.append(c)
        idx += 1

    def split_copy(src, dst, rows, chunk):
        for r0 in range(0, rows, chunk):
            start_copy(src.at[pl.ds(r0, chunk)], dst.at[pl.ds(r0, chunk)])

    split_copy(p_hbm, p_v, f_num * n_p, in_chunk)
    split_copy(o_hbm, o_v, f_num * n_o, in_chunk)
    split_copy(w_hbm, w_v, d, in_chunk // 2)
    start_copy(b_hbm, b_v)
    for c in copies:
        c.wait()

    wp = w_v[:, :d]
    wo = w_v[:, d:]
    a = jax.lax.dot_general(p_v[:], wp, (((1,), (1,)), ((), ())),
                            preferred_element_type=jnp.float32)
    b_mat = jax.lax.dot_general(o_v[:], wo, (((1,), (1,)), ((), ())),
                                preferred_element_type=jnp.float32)
    b_max = jnp.max(b_mat.reshape(f_num, n_o, d), axis=1)
    b_rep = jnp.broadcast_to(b_max[:, None, :], (f_num, n_p, d))
    r_v[:] = a + b_rep.reshape(f_num * n_p, d) + b_v[:]

    out_copies = []
    for r0 in range(0, f_num * n_p, out_chunk):
        c = pltpu.make_async_copy(r_v.at[pl.ds(r0, out_chunk)],
                                  out_hbm.at[pl.ds(r0, out_chunk)],
                                  sem.at[idx])
        c.start()
        out_copies.append(c)
        idx += 1
    for c in out_copies:
        c.wait()


def kernel(person_features, other_features, person_boxes, other_boxes,
           is_person, W, b):
    f_num, n_p = person_boxes.shape[0], person_boxes.shape[1]
    n_o = other_boxes.shape[1]
    d = person_features.shape[1]
    d_out = W.shape[0]
    person = person_features.reshape(f_num * n_p, d)
    other = other_features.reshape(f_num * n_o, d)

    in_chunk = 256
    out_chunk = 128
    n_sems = ((f_num * n_p) // in_chunk + (f_num * n_o) // in_chunk
              + d // (in_chunk // 2) + 1 + (f_num * n_p) // out_chunk)

    hbm = pltpu.MemorySpace.HBM
    out = pl.pallas_call(
        functools.partial(_relation_kernel, f_num=f_num, n_p=n_p, n_o=n_o,
                          d=d, in_chunk=in_chunk, out_chunk=out_chunk),
        in_specs=[pl.BlockSpec(memory_space=hbm)] * 4,
        out_specs=pl.BlockSpec(memory_space=hbm),
        out_shape=jax.ShapeDtypeStruct((f_num * n_p, d_out), jnp.float32),
        scratch_shapes=[
            pltpu.VMEM((f_num * n_p, d), jnp.float32),
            pltpu.VMEM((f_num * n_o, d), jnp.float32),
            pltpu.VMEM((d_out, 2 * d), jnp.float32),
            pltpu.VMEM((1, d_out), jnp.float32),
            pltpu.VMEM((f_num * n_p, d_out), jnp.float32),
            pltpu.SemaphoreType.DMA((n_sems,)),
        ],
    )(person, other, W, b.reshape(1, d_out))
    return out[:, :, None, None]


# speedup vs baseline: 1.0130x; 1.0130x over previous
import functools

import jax
import jax.numpy as jnp
from jax.experimental import pallas as pl
from jax.experimental.pallas import tpu as pltpu


def _relation_kernel(p_hbm, o_hbm, w_hbm, b_hbm, out_hbm,
                     p_v, o_v, w_v, b_v, r_v, sem, *,
                     f_num, n_p, n_o, d):
    copies = []
    idx = 0

    def start_copy(src, dst):
        nonlocal idx
        c = pltpu.make_async_copy(src, dst, sem.at[idx])
        c.start()
        copies.append(c)
        idx += 1

    n_pp = f_num * n_p
    n_oo = f_num * n_o
    start_copy(p_hbm, p_v)
    start_copy(o_hbm.at[pl.ds(0, n_oo // 2)], o_v.at[pl.ds(0, n_oo // 2)])
    start_copy(o_hbm.at[pl.ds(n_oo // 2, n_oo // 2)],
               o_v.at[pl.ds(n_oo // 2, n_oo // 2)])
    start_copy(w_hbm.at[pl.ds(0, d // 2)], w_v.at[pl.ds(0, d // 2)])
    start_copy(w_hbm.at[pl.ds(d // 2, d // 2)],
               w_v.at[pl.ds(d // 2, d // 2)])
    start_copy(b_hbm, b_v)
    for c in copies:
        c.wait()

    wp = w_v[:, :d]
    wo = w_v[:, d:]
    a = jax.lax.dot_general(p_v[:], wp, (((1,), (1,)), ((), ())),
                            preferred_element_type=jnp.float32)
    b_mat = jax.lax.dot_general(o_v[:], wo, (((1,), (1,)), ((), ())),
                                preferred_element_type=jnp.float32)
    b_max = jnp.max(b_mat.reshape(f_num, n_o, d), axis=1)
    b_rep = jnp.broadcast_to(b_max[:, None, :], (f_num, n_p, d))
    r_v[:] = a + b_rep.reshape(n_pp, d) + b_v[:]

    out_copies = []
    for r0 in range(0, n_pp, n_pp // 4):
        c = pltpu.make_async_copy(r_v.at[pl.ds(r0, n_pp // 4)],
                                  out_hbm.at[pl.ds(r0, n_pp // 4)],
                                  sem.at[idx])
        c.start()
        out_copies.append(c)
        idx += 1
    for c in out_copies:
        c.wait()


def kernel(person_features, other_features, person_boxes, other_boxes,
           is_person, W, b):
    f_num, n_p = person_boxes.shape[0], person_boxes.shape[1]
    n_o = other_boxes.shape[1]
    d = person_features.shape[1]
    d_out = W.shape[0]
    person = person_features.reshape(f_num * n_p, d)
    other = other_features.reshape(f_num * n_o, d)

    hbm = pltpu.MemorySpace.HBM
    out = pl.pallas_call(
        functools.partial(_relation_kernel, f_num=f_num, n_p=n_p, n_o=n_o, d=d),
        in_specs=[pl.BlockSpec(memory_space=hbm)] * 4,
        out_specs=pl.BlockSpec(memory_space=hbm),
        out_shape=jax.ShapeDtypeStruct((f_num * n_p, d_out), jnp.float32),
        scratch_shapes=[
            pltpu.VMEM((f_num * n_p, d), jnp.float32),
            pltpu.VMEM((f_num * n_o, d), jnp.float32),
            pltpu.VMEM((d_out, 2 * d), jnp.float32),
            pltpu.VMEM((1, d_out), jnp.float32),
            pltpu.VMEM((f_num * n_p, d_out), jnp.float32),
            pltpu.SemaphoreType.DMA((10,)),
        ],
    )(person, other, W, b.reshape(1, d_out))
    return out[:, :, None, None]
